# baseline (speedup 1.0000x reference)
import jax
import jax.numpy as jnp
from jax.experimental import pallas as pl
from jax.experimental.pallas import tpu as pltpu

GAMMA_ = 0.99
NEG_INF = float("-inf")
B_ = 128
A_ = 100000

WR_ = 16384
NFULL_ = A_ // WR_
TAIL_ = A_ - NFULL_ * WR_
RING_ = 4
AHEAD_ = 2


def _argmax_body(nt_ref, idx_ref, bufs, tailbuf, sems):
    def dma(b, slot):
        return pltpu.make_async_copy(
            nt_ref.at[pl.ds(b * WR_, WR_), :], bufs.at[slot], sems.at[slot])

    for b in range(min(RING_, NFULL_)):
        dma(b, b % RING_).start()

    rmax = jnp.full((1, B_), NEG_INF, jnp.float32)
    ridx = jnp.zeros((1, B_), jnp.int32)

    def step(v, base, rmax, ridx):
        bidx = jnp.argmax(v, axis=0, keepdims=True).astype(jnp.int32) + base
        bmax = jnp.max(v, axis=0, keepdims=True)
        upd = bmax > rmax
        return (jnp.where(upd, bmax, rmax), jnp.where(upd, bidx, ridx))

    for b in range(NFULL_):
        dma(b, b % RING_).wait()
        rmax, ridx = step(bufs[b % RING_], b * WR_, rmax, ridx)
        if b + RING_ < NFULL_:
            dma(b + RING_, b % RING_).start()

    tcopy = pltpu.make_async_copy(
        nt_ref.at[pl.ds(NFULL_ * WR_, TAIL_), :], tailbuf, sems.at[0])
    tcopy.start()
    tcopy.wait()
    rmax, ridx = step(tailbuf[...], NFULL_ * WR_, rmax, ridx)

    idx_ref[...] = ridx


def _argmax_call(nt):
    return pl.pallas_call(
        _argmax_body,
        in_specs=[pl.BlockSpec(memory_space=pltpu.MemorySpace.HBM)],
        out_specs=pl.BlockSpec(memory_space=pltpu.MemorySpace.VMEM),
        out_shape=jax.ShapeDtypeStruct((1, B_), jnp.int32),
        scratch_shapes=[
            pltpu.VMEM((RING_, WR_, B_), jnp.float32),
            pltpu.VMEM((TAIL_, B_), jnp.float32),
            pltpu.SemaphoreType.DMA((RING_,)),
        ],
    )(nt)


def _gather_body(tt_ref, qt_ref, ba_s, a_s, r_ref, d_ref,
                 td_ref, loss_ref, tstage, qstage, sems):
    reads = []
    for i in range(B_):
        cp = pltpu.make_async_copy(
            tt_ref.at[pl.ds(ba_s[0, i], 1), :], tstage.at[pl.ds(i, 1)],
            sems.at[i % 8])
        cp.start()
        reads.append(cp)
        cp2 = pltpu.make_async_copy(
            qt_ref.at[pl.ds(a_s[0, i], 1), :], qstage.at[pl.ds(i, 1)],
            sems.at[i % 8])
        cp2.start()
        reads.append(cp2)
    for cp in reads:
        cp.wait()

    lane = jax.lax.broadcasted_iota(jnp.int32, (B_, B_), 1)
    row = jax.lax.broadcasted_iota(jnp.int32, (B_, B_), 0)
    diag = lane == row
    tval = jnp.sum(jnp.where(diag, tstage[...], 0.0), axis=0, keepdims=True)
    qv = jnp.sum(jnp.where(diag, qstage[...], 0.0), axis=0, keepdims=True)
    td = r_ref[...] + (1.0 - d_ref[...]) * GAMMA_ * tval
    td_ref[...] = td
    loss_ref[0, 0] = jnp.sum((qv - td) ** 2) * (1.0 / (B_ * A_))


def _gather_call(tt, qt, ba, a2, r2, d2):
    return pl.pallas_call(
        _gather_body,
        in_specs=[
            pl.BlockSpec(memory_space=pltpu.MemorySpace.HBM),
            pl.BlockSpec(memory_space=pltpu.MemorySpace.HBM),
            pl.BlockSpec(memory_space=pltpu.MemorySpace.SMEM),
            pl.BlockSpec(memory_space=pltpu.MemorySpace.SMEM),
            pl.BlockSpec(memory_space=pltpu.MemorySpace.VMEM),
            pl.BlockSpec(memory_space=pltpu.MemorySpace.VMEM),
        ],
        out_specs=[
            pl.BlockSpec(memory_space=pltpu.MemorySpace.VMEM),
            pl.BlockSpec(memory_space=pltpu.MemorySpace.SMEM),
        ],
        out_shape=[
            jax.ShapeDtypeStruct((1, B_), jnp.float32),
            jax.ShapeDtypeStruct((1, 1), jnp.float32),
        ],
        scratch_shapes=[
            pltpu.VMEM((B_, B_), jnp.float32),
            pltpu.VMEM((B_, B_), jnp.float32),
            pltpu.SemaphoreType.DMA((8,)),
        ],
    )(tt, qt, ba, a2, r2, d2)


def _copy_body(qt_ref, a_ref, td_ref, yt_ref, bufs, tailbuf, rsems, wsems):
    a_b = a_ref[...]
    td_b = td_ref[...]

    def rd(b, slot):
        return pltpu.make_async_copy(
            qt_ref.at[pl.ds(b * WR_, WR_), :], bufs.at[slot], rsems.at[slot])

    def wr(b, slot):
        return pltpu.make_async_copy(
            bufs.at[slot], yt_ref.at[pl.ds(b * WR_, WR_), :], wsems.at[slot])

    def patch(buf, base):
        ids = jax.lax.broadcasted_iota(jnp.int32, buf.shape, 0) + base
        buf[...] = jnp.where(ids == a_b, td_b, buf[...])

    wr_waited = -1
    for b in range(min(AHEAD_, NFULL_)):
        rd(b, b % RING_).start()

    for b in range(NFULL_):
        rd(b, b % RING_).wait()
        patch(bufs.at[b % RING_], b * WR_)
        wr(b, b % RING_).start()
        nxt = b + AHEAD_
        if nxt < NFULL_:
            prev = nxt - RING_
            if prev >= 0:
                wr(prev, prev % RING_).wait()
                wr_waited = prev
            rd(nxt, nxt % RING_).start()

    trd = pltpu.make_async_copy(
        qt_ref.at[pl.ds(NFULL_ * WR_, TAIL_), :], tailbuf, rsems.at[0])
    trd.start()
    trd.wait()
    patch(tailbuf, NFULL_ * WR_)
    twr = pltpu.make_async_copy(
        tailbuf, yt_ref.at[pl.ds(NFULL_ * WR_, TAIL_), :], wsems.at[0])
    twr.start()
    for b in range(wr_waited + 1, NFULL_):
        wr(b, b % RING_).wait()
    twr.wait()


def _copy_call(qt, a2, td):
    return pl.pallas_call(
        _copy_body,
        in_specs=[
            pl.BlockSpec(memory_space=pltpu.MemorySpace.HBM),
            pl.BlockSpec(memory_space=pltpu.MemorySpace.VMEM),
            pl.BlockSpec(memory_space=pltpu.MemorySpace.VMEM),
        ],
        out_specs=pl.BlockSpec(memory_space=pltpu.MemorySpace.HBM),
        out_shape=jax.ShapeDtypeStruct((A_, B_), jnp.float32),
        scratch_shapes=[
            pltpu.VMEM((RING_, WR_, B_), jnp.float32),
            pltpu.VMEM((TAIL_, B_), jnp.float32),
            pltpu.SemaphoreType.DMA((RING_,)),
            pltpu.SemaphoreType.DMA((RING_,)),
        ],
    )(qt, a2, td)


def kernel(q_values, target_q_values, next_q_values, actions, rewards, dones):
    B, A = q_values.shape
    assert (B, A) == (B_, A_)
    qt = q_values.T
    tt = target_q_values.T
    nt = next_q_values.T
    a2 = actions.reshape(1, B).astype(jnp.int32)
    r2 = rewards.reshape(1, B).astype(jnp.float32)
    d2 = dones.reshape(1, B).astype(jnp.float32)

    ba = _argmax_call(nt)
    td, loss = _gather_call(tt, qt, ba, a2, r2, d2)
    yt = _copy_call(qt, a2, td)
    return yt.T, td.reshape(B), loss.reshape(())


# speedup vs baseline: 1.0201x; 1.0201x over previous
import jax
import jax.numpy as jnp
from jax.experimental import pallas as pl
from jax.experimental.pallas import tpu as pltpu

GAMMA_ = 0.99
NEG_INF = float("-inf")
B_ = 128
A_ = 100000

WR_ = 8192
NFULL_ = A_ // WR_
TAIL_ = A_ - NFULL_ * WR_
RING_ = 6
AHEAD_ = 3


def _argmax_body(nt_ref, idx_ref, bufs, tailbuf, sems):
    def dma(b, slot):
        return pltpu.make_async_copy(
            nt_ref.at[pl.ds(b * WR_, WR_), :], bufs.at[slot], sems.at[slot])

    for b in range(min(RING_, NFULL_)):
        dma(b, b % RING_).start()

    rmax = jnp.full((1, B_), NEG_INF, jnp.float32)
    ridx = jnp.zeros((1, B_), jnp.int32)

    def step(v, base, rmax, ridx):
        bidx = jnp.argmax(v, axis=0, keepdims=True).astype(jnp.int32) + base
        bmax = jnp.max(v, axis=0, keepdims=True)
        upd = bmax > rmax
        return (jnp.where(upd, bmax, rmax), jnp.where(upd, bidx, ridx))

    for b in range(NFULL_):
        dma(b, b % RING_).wait()
        rmax, ridx = step(bufs[b % RING_], b * WR_, rmax, ridx)
        if b + RING_ < NFULL_:
            dma(b + RING_, b % RING_).start()

    tcopy = pltpu.make_async_copy(
        nt_ref.at[pl.ds(NFULL_ * WR_, TAIL_), :], tailbuf, sems.at[0])
    tcopy.start()
    tcopy.wait()
    rmax, ridx = step(tailbuf[...], NFULL_ * WR_, rmax, ridx)

    idx_ref[...] = ridx


def _argmax_call(nt):
    return pl.pallas_call(
        _argmax_body,
        in_specs=[pl.BlockSpec(memory_space=pltpu.MemorySpace.HBM)],
        out_specs=pl.BlockSpec(memory_space=pltpu.MemorySpace.VMEM),
        out_shape=jax.ShapeDtypeStruct((1, B_), jnp.int32),
        scratch_shapes=[
            pltpu.VMEM((RING_, WR_, B_), jnp.float32),
            pltpu.VMEM((TAIL_, B_), jnp.float32),
            pltpu.SemaphoreType.DMA((RING_,)),
        ],
    )(nt)


def _gather_body(tt_ref, qt_ref, ba_s, a_s, r_ref, d_ref,
                 td_ref, loss_ref, tstage, qstage, sems):
    reads = []
    for i in range(B_):
        cp = pltpu.make_async_copy(
            tt_ref.at[pl.ds(ba_s[0, i], 1), :], tstage.at[pl.ds(i, 1)],
            sems.at[i % 8])
        cp.start()
        reads.append(cp)
        cp2 = pltpu.make_async_copy(
            qt_ref.at[pl.ds(a_s[0, i], 1), :], qstage.at[pl.ds(i, 1)],
            sems.at[i % 8])
        cp2.start()
        reads.append(cp2)
    for cp in reads:
        cp.wait()

    lane = jax.lax.broadcasted_iota(jnp.int32, (B_, B_), 1)
    row = jax.lax.broadcasted_iota(jnp.int32, (B_, B_), 0)
    diag = lane == row
    tval = jnp.sum(jnp.where(diag, tstage[...], 0.0), axis=0, keepdims=True)
    qv = jnp.sum(jnp.where(diag, qstage[...], 0.0), axis=0, keepdims=True)
    td = r_ref[...] + (1.0 - d_ref[...]) * GAMMA_ * tval
    td_ref[...] = td
    loss_ref[0, 0] = jnp.sum((qv - td) ** 2) * (1.0 / (B_ * A_))


def _gather_call(tt, qt, ba, a2, r2, d2):
    return pl.pallas_call(
        _gather_body,
        in_specs=[
            pl.BlockSpec(memory_space=pltpu.MemorySpace.HBM),
            pl.BlockSpec(memory_space=pltpu.MemorySpace.HBM),
            pl.BlockSpec(memory_space=pltpu.MemorySpace.SMEM),
            pl.BlockSpec(memory_space=pltpu.MemorySpace.SMEM),
            pl.BlockSpec(memory_space=pltpu.MemorySpace.VMEM),
            pl.BlockSpec(memory_space=pltpu.MemorySpace.VMEM),
        ],
        out_specs=[
            pl.BlockSpec(memory_space=pltpu.MemorySpace.VMEM),
            pl.BlockSpec(memory_space=pltpu.MemorySpace.SMEM),
        ],
        out_shape=[
            jax.ShapeDtypeStruct((1, B_), jnp.float32),
            jax.ShapeDtypeStruct((1, 1), jnp.float32),
        ],
        scratch_shapes=[
            pltpu.VMEM((B_, B_), jnp.float32),
            pltpu.VMEM((B_, B_), jnp.float32),
            pltpu.SemaphoreType.DMA((8,)),
        ],
    )(tt, qt, ba, a2, r2, d2)


def _copy_body(qt_ref, a_ref, td_ref, yt_ref, bufs, tailbuf, rsems, wsems):
    a_b = a_ref[...]
    td_b = td_ref[...]

    def rd(b, slot):
        return pltpu.make_async_copy(
            qt_ref.at[pl.ds(b * WR_, WR_), :], bufs.at[slot], rsems.at[slot])

    def wr(b, slot):
        return pltpu.make_async_copy(
            bufs.at[slot], yt_ref.at[pl.ds(b * WR_, WR_), :], wsems.at[slot])

    def patch(buf, base):
        ids = jax.lax.broadcasted_iota(jnp.int32, buf.shape, 0) + base
        buf[...] = jnp.where(ids == a_b, td_b, buf[...])

    wr_waited = -1
    for b in range(min(AHEAD_, NFULL_)):
        rd(b, b % RING_).start()

    for b in range(NFULL_):
        rd(b, b % RING_).wait()
        patch(bufs.at[b % RING_], b * WR_)
        wr(b, b % RING_).start()
        nxt = b + AHEAD_
        if nxt < NFULL_:
            prev = nxt - RING_
            if prev >= 0:
                wr(prev, prev % RING_).wait()
                wr_waited = prev
            rd(nxt, nxt % RING_).start()

    trd = pltpu.make_async_copy(
        qt_ref.at[pl.ds(NFULL_ * WR_, TAIL_), :], tailbuf, rsems.at[0])
    trd.start()
    trd.wait()
    patch(tailbuf, NFULL_ * WR_)
    twr = pltpu.make_async_copy(
        tailbuf, yt_ref.at[pl.ds(NFULL_ * WR_, TAIL_), :], wsems.at[0])
    twr.start()
    for b in range(wr_waited + 1, NFULL_):
        wr(b, b % RING_).wait()
    twr.wait()


def _copy_call(qt, a2, td):
    return pl.pallas_call(
        _copy_body,
        in_specs=[
            pl.BlockSpec(memory_space=pltpu.MemorySpace.HBM),
            pl.BlockSpec(memory_space=pltpu.MemorySpace.VMEM),
            pl.BlockSpec(memory_space=pltpu.MemorySpace.VMEM),
        ],
        out_specs=pl.BlockSpec(memory_space=pltpu.MemorySpace.HBM),
        out_shape=jax.ShapeDtypeStruct((A_, B_), jnp.float32),
        scratch_shapes=[
            pltpu.VMEM((RING_, WR_, B_), jnp.float32),
            pltpu.VMEM((TAIL_, B_), jnp.float32),
            pltpu.SemaphoreType.DMA((RING_,)),
            pltpu.SemaphoreType.DMA((RING_,)),
        ],
    )(qt, a2, td)


def kernel(q_values, target_q_values, next_q_values, actions, rewards, dones):
    B, A = q_values.shape
    assert (B, A) == (B_, A_)
    qt = q_values.T
    tt = target_q_values.T
    nt = next_q_values.T
    a2 = actions.reshape(1, B).astype(jnp.int32)
    r2 = rewards.reshape(1, B).astype(jnp.float32)
    d2 = dones.reshape(1, B).astype(jnp.float32)

    ba = _argmax_call(nt)
    td, loss = _gather_call(tt, qt, ba, a2, r2, d2)
    yt = _copy_call(qt, a2, td)
    return yt.T, td.reshape(B), loss.reshape(())
